# baseline (speedup 1.0000x reference)
import jax
import jax.numpy as jnp
from jax.experimental import pallas as pl

_S = 64.0
_M = 0.4

_BLOCK_R = 1600


def _body(labels_ref, x_ref, o_ref):
    i = pl.program_id(0)
    br, b = x_ref.shape
    rows = i * _BLOCK_R + jax.lax.broadcasted_iota(jnp.int32, (br, b), 0)
    lab = labels_ref[...]
    x = x_ref[...]
    o_ref[...] = (x - jnp.where(rows == lab, _M, 0.0)) * _S


def kernel(logits, norms, labels):
    del norms
    b, c = logits.shape
    lt = logits.T
    labels_row = labels.astype(jnp.int32).reshape(1, b)
    out_t = pl.pallas_call(
        _body,
        grid=(pl.cdiv(c, _BLOCK_R),),
        in_specs=[
            pl.BlockSpec((1, b), lambda i: (0, 0)),
            pl.BlockSpec((_BLOCK_R, b), lambda i: (i, 0)),
        ],
        out_specs=pl.BlockSpec((_BLOCK_R, b), lambda i: (i, 0)),
        out_shape=jax.ShapeDtypeStruct((c, b), jnp.float32),
    )(labels_row, lt)
    return out_t.T


# speedup vs baseline: 1.0079x; 1.0079x over previous
import jax
import jax.numpy as jnp
from jax.experimental import pallas as pl

_S = 64.0
_M = 0.4

_BLOCK_R = 3200


def _body(labels_ref, x_ref, o_ref):
    i = pl.program_id(0)
    br, b = x_ref.shape
    rows = i * _BLOCK_R + jax.lax.broadcasted_iota(jnp.int32, (br, b), 0)
    lab = labels_ref[...]
    x = x_ref[...]
    o_ref[...] = (x - jnp.where(rows == lab, _M, 0.0)) * _S


def kernel(logits, norms, labels):
    del norms
    b, c = logits.shape
    lt = logits.T
    labels_row = labels.astype(jnp.int32).reshape(1, b)
    out_t = pl.pallas_call(
        _body,
        grid=(pl.cdiv(c, _BLOCK_R),),
        in_specs=[
            pl.BlockSpec((1, b), lambda i: (0, 0)),
            pl.BlockSpec((_BLOCK_R, b), lambda i: (i, 0)),
        ],
        out_specs=pl.BlockSpec((_BLOCK_R, b), lambda i: (i, 0)),
        out_shape=jax.ShapeDtypeStruct((c, b), jnp.float32),
    )(labels_row, lt)
    return out_t.T
